# baseline (speedup 1.0000x reference)
import functools

import jax
import jax.numpy as jnp
from jax import lax
from jax.experimental import pallas as pl
from jax.experimental.pallas import tpu as pltpu
from jax.experimental.pallas import tpu_sc as plsc

_NC = 2
_NS = 16
_NW = _NC * _NS

_BM = 64


def _dispatch_meta(expert_ids, num_experts, cap, bm):
    nt, tk = expert_ids.shape
    p = nt * tk
    flat_e = expert_ids.reshape(-1).astype(jnp.int32)

    onehot = (flat_e[:, None] == jnp.arange(num_experts, dtype=jnp.int32)
              ).astype(jnp.int32)
    cum = jnp.cumsum(onehot, axis=0)
    rank = jnp.take_along_axis(cum, flat_e[:, None], axis=1)[:, 0] - 1
    counts = cum[-1]

    padded = ((counts + bm - 1) // bm) * bm
    pad_end = jnp.cumsum(padded)
    pad_start = pad_end - padded

    slot = pad_start[flat_e] + rank

    nb = cap // bm
    block_rows = jnp.arange(nb, dtype=jnp.int32)[:, None] * bm
    block_expert = jnp.minimum(
        jnp.sum((block_rows >= pad_end[None, :]).astype(jnp.int32), axis=1),
        num_experts - 1).astype(jnp.int32)
    return slot, block_expert


def _sc_dispatch(x, slot_3d, cap, tk):
    nt, d = x.shape
    per_w = nt // _NW
    mesh = plsc.VectorSubcoreMesh(core_axis_name="c", subcore_axis_name="s")

    scratch = ([pltpu.VMEM((per_w, d), x.dtype),
                pltpu.VMEM((tk, per_w), jnp.int32)]
               + [pltpu.SemaphoreType.DMA for _ in range(tk + 1)])

    @functools.partial(
        pl.kernel,
        mesh=mesh,
        out_type=jax.ShapeDtypeStruct((cap, d), x.dtype),
        scratch_types=scratch,
    )
    def dispatch_kernel(x_hbm, idx_hbm, out_hbm, xbuf, idx_v, *sems):
        wid = lax.axis_index("s") * _NC + lax.axis_index("c")
        base = wid * per_w
        xcp = pltpu.async_copy(x_hbm.at[pl.ds(base, per_w)], xbuf, sems[tk])
        pltpu.sync_copy(idx_hbm.at[wid], idx_v)
        xcp.wait()
        cps = [pltpu.async_copy(xbuf, out_hbm.at[idx_v.at[k]], sems[k])
               for k in range(tk)]
        for cp in cps:
            cp.wait()

    return dispatch_kernel(x, slot_3d)


def _tc_moe_mlp(xs, gate_w, up_w, down_w, block_expert, cap, bm):
    e, d, f = gate_w.shape
    nb = cap // bm

    def mlp_kernel(be_ref, xs_ref, g_ref, u_ref, d_ref, out_ref):
        xb = xs_ref[...]
        go = jnp.dot(xb, g_ref[0], preferred_element_type=jnp.float32)
        uo = jnp.dot(xb, u_ref[0], preferred_element_type=jnp.float32)
        act = (go * jax.nn.sigmoid(go)) * uo
        out_ref[...] = jnp.dot(act, d_ref[0],
                               preferred_element_type=jnp.float32)

    grid_spec = pltpu.PrefetchScalarGridSpec(
        num_scalar_prefetch=1,
        grid=(nb,),
        in_specs=[
            pl.BlockSpec((bm, d), lambda b, be: (b, 0)),
            pl.BlockSpec((1, d, f), lambda b, be: (be[b], 0, 0)),
            pl.BlockSpec((1, d, f), lambda b, be: (be[b], 0, 0)),
            pl.BlockSpec((1, f, d), lambda b, be: (be[b], 0, 0)),
        ],
        out_specs=pl.BlockSpec((bm, d), lambda b, be: (b, 0)),
    )
    return pl.pallas_call(
        mlp_kernel,
        grid_spec=grid_spec,
        out_shape=jax.ShapeDtypeStruct((cap, d), jnp.float32),
        compiler_params=pltpu.CompilerParams(
            dimension_semantics=("arbitrary",)),
    )(block_expert, xs, gate_w, up_w, down_w)


def _sc_combine(y, pos_cols, w_cols, nt, d, tk):
    per_w = nt // _NW
    chunk = 16 if per_w % 16 == 0 else per_w
    n_chunks = per_w // chunk
    mesh = plsc.VectorSubcoreMesh(core_axis_name="c", subcore_axis_name="s")

    scratch = []
    for _ in range(tk):
        scratch.append(pltpu.VMEM((per_w,), jnp.int32))
        scratch.append(pltpu.VMEM((per_w, 16), jnp.float32))
        scratch.append(pltpu.VMEM((2, chunk, d), jnp.float32))
        scratch.append(pltpu.SemaphoreType.DMA)
        scratch.append(pltpu.SemaphoreType.DMA)
    scratch.append(pltpu.SemaphoreType.DMA)
    scratch.append(pltpu.SemaphoreType.DMA)

    @functools.partial(
        pl.kernel,
        mesh=mesh,
        out_type=jax.ShapeDtypeStruct((nt, d), jnp.float32),
        scratch_types=scratch,
    )
    def combine_kernel(y_hbm, *rest):
        pos_hbm = rest[:tk]
        w_hbm = rest[tk:2 * tk]
        out_hbm = rest[2 * tk]
        sc = rest[2 * tk + 1:]
        idx_v = sc[0:5 * tk:5]
        w_v = sc[1:5 * tk:5]
        buf_v = sc[2:5 * tk:5]
        gsem = [sc[5 * k + 3:5 * k + 5] for k in range(tk)]
        wsem = sc[5 * tk:]
        wid = lax.axis_index("s") * _NC + lax.axis_index("c")
        base = wid * per_w
        for k in range(tk):
            pltpu.sync_copy(pos_hbm[k].at[pl.ds(base, per_w)], idx_v[k])
            pltpu.sync_copy(w_hbm[k].at[pl.ds(base, per_w)], w_v[k])

        def start_wave(c):
            par = c % 2
            return [pltpu.async_copy(
                y_hbm.at[idx_v[k].at[pl.ds(c * chunk, chunk)]],
                buf_v[k].at[par], gsem[k][par]) for k in range(tk)]

        def add_wave(c):
            par = c % 2

            def row_body(r, _):
                wk = [w_v[k][c * chunk + r, :] for k in range(tk)]

                def col_body(ci, _):
                    off = ci * 64
                    for s in range(4):
                        sl = pl.ds(off + s * 16, 16)
                        acc = buf_v[0][par, r, sl] * wk[0]
                        for k in range(1, tk):
                            acc = acc + buf_v[k][par, r, sl] * wk[k]
                        buf_v[0][par, r, sl] = acc
                    return 0

                return lax.fori_loop(0, d // 64, col_body, 0)

            lax.fori_loop(0, chunk, row_body, 0)
            return pltpu.async_copy(
                buf_v[0].at[par], out_hbm.at[pl.ds(base + c * chunk, chunk)],
                wsem[par])

        gq = [None] * n_chunks
        wq = [None] * n_chunks
        gq[0] = start_wave(0)
        for c in range(n_chunks):
            if c + 1 < n_chunks:
                if c >= 1:
                    wq[c - 1].wait()
                gq[c + 1] = start_wave(c + 1)
            for cp in gq[c]:
                cp.wait()
            wq[c] = add_wave(c)
        for c in range(max(0, n_chunks - 2), n_chunks):
            wq[c].wait()

    return combine_kernel(y, *pos_cols, *w_cols)


def kernel(x, expert_ids, expert_weights, gate_weights, up_weights,
           down_weights):
    nt, d = x.shape
    tk = expert_ids.shape[1]
    num_experts = gate_weights.shape[0]
    p = nt * tk
    cap = p + num_experts * _BM

    slot, block_expert = _dispatch_meta(expert_ids, num_experts, cap, _BM)

    per_w = nt // _NW
    slot_3d = slot.reshape(_NW, per_w, tk).transpose(0, 2, 1).copy()
    pos2d = slot.reshape(nt, tk)
    pos_cols = [pos2d[:, k].copy() for k in range(tk)]
    w_cols = [jnp.broadcast_to(expert_weights[:, k][:, None], (nt, 16)).copy()
              for k in range(tk)]

    xs = _sc_dispatch(x, slot_3d, cap, tk)
    y = _tc_moe_mlp(xs, gate_weights, up_weights, down_weights, block_expert,
                    cap, _BM)
    return _sc_combine(y, pos_cols, w_cols, nt, d, tk)


# speedup vs baseline: 1.4365x; 1.4365x over previous
import functools

import jax
import jax.numpy as jnp
from jax import lax
from jax.experimental import pallas as pl
from jax.experimental.pallas import tpu as pltpu
from jax.experimental.pallas import tpu_sc as plsc

_NC = 2
_NS = 16
_NW = _NC * _NS

_BM = 128


def _dispatch_meta(expert_ids, num_experts, cap, bm):
    nt, tk = expert_ids.shape
    p = nt * tk
    flat_e = expert_ids.reshape(-1).astype(jnp.int32)

    onehot = (flat_e[:, None] == jnp.arange(num_experts, dtype=jnp.int32)
              ).astype(jnp.int32)
    cum = jnp.cumsum(onehot, axis=0)
    rank = jnp.take_along_axis(cum, flat_e[:, None], axis=1)[:, 0] - 1
    counts = cum[-1]

    padded = ((counts + bm - 1) // bm) * bm
    pad_end = jnp.cumsum(padded)
    pad_start = pad_end - padded

    slot = pad_start[flat_e] + rank

    nb = cap // bm
    block_rows = jnp.arange(nb, dtype=jnp.int32)[:, None] * bm
    block_expert = jnp.minimum(
        jnp.sum((block_rows >= pad_end[None, :]).astype(jnp.int32), axis=1),
        num_experts - 1).astype(jnp.int32)
    return slot, block_expert


def _sc_dispatch(x, slot_3d, cap, tk):
    nt, d = x.shape
    per_w = nt // _NW
    mesh = plsc.VectorSubcoreMesh(core_axis_name="c", subcore_axis_name="s")

    scratch = ([pltpu.VMEM((per_w, d), x.dtype),
                pltpu.VMEM((tk, per_w), jnp.int32)]
               + [pltpu.SemaphoreType.DMA for _ in range(tk + 1)])

    @functools.partial(
        pl.kernel,
        mesh=mesh,
        out_type=jax.ShapeDtypeStruct((cap, d), x.dtype),
        scratch_types=scratch,
    )
    def dispatch_kernel(x_hbm, idx_hbm, out_hbm, xbuf, idx_v, *sems):
        wid = lax.axis_index("s") * _NC + lax.axis_index("c")
        base = wid * per_w
        xcp = pltpu.async_copy(x_hbm.at[pl.ds(base, per_w)], xbuf, sems[tk])
        pltpu.sync_copy(idx_hbm.at[wid], idx_v)
        xcp.wait()
        cps = [pltpu.async_copy(xbuf, out_hbm.at[idx_v.at[k]], sems[k])
               for k in range(tk)]
        for cp in cps:
            cp.wait()

    return dispatch_kernel(x, slot_3d)


def _tc_moe_mlp(xs, gate_w, up_w, down_w, block_expert, cap, bm):
    e, d, f = gate_w.shape
    nb = cap // bm

    def mlp_kernel(be_ref, xs_ref, g_ref, u_ref, d_ref, out_ref):
        xb = xs_ref[...]
        go = jnp.dot(xb, g_ref[0], preferred_element_type=jnp.float32)
        uo = jnp.dot(xb, u_ref[0], preferred_element_type=jnp.float32)
        act = (go * jax.nn.sigmoid(go)) * uo
        out_ref[...] = jnp.dot(act, d_ref[0],
                               preferred_element_type=jnp.float32)

    grid_spec = pltpu.PrefetchScalarGridSpec(
        num_scalar_prefetch=1,
        grid=(nb,),
        in_specs=[
            pl.BlockSpec((bm, d), lambda b, be: (b, 0)),
            pl.BlockSpec((1, d, f), lambda b, be: (be[b], 0, 0)),
            pl.BlockSpec((1, d, f), lambda b, be: (be[b], 0, 0)),
            pl.BlockSpec((1, f, d), lambda b, be: (be[b], 0, 0)),
        ],
        out_specs=pl.BlockSpec((bm, d), lambda b, be: (b, 0)),
    )
    return pl.pallas_call(
        mlp_kernel,
        grid_spec=grid_spec,
        out_shape=jax.ShapeDtypeStruct((cap, d), jnp.float32),
        compiler_params=pltpu.CompilerParams(
            dimension_semantics=("arbitrary",)),
    )(block_expert, xs, gate_w, up_w, down_w)


def _sc_combine(y, pos_cols, w_cols, nt, d, tk):
    per_w = nt // _NW
    chunk = 16 if per_w % 16 == 0 else per_w
    n_chunks = per_w // chunk
    mesh = plsc.VectorSubcoreMesh(core_axis_name="c", subcore_axis_name="s")

    scratch = []
    for _ in range(tk):
        scratch.append(pltpu.VMEM((per_w,), jnp.int32))
        scratch.append(pltpu.VMEM((per_w, 16), jnp.float32))
        scratch.append(pltpu.VMEM((2, chunk, d), jnp.float32))
        scratch.append(pltpu.SemaphoreType.DMA)
        scratch.append(pltpu.SemaphoreType.DMA)
    scratch.append(pltpu.SemaphoreType.DMA)
    scratch.append(pltpu.SemaphoreType.DMA)

    @functools.partial(
        pl.kernel,
        mesh=mesh,
        out_type=jax.ShapeDtypeStruct((nt, d), jnp.float32),
        scratch_types=scratch,
    )
    def combine_kernel(y_hbm, *rest):
        pos_hbm = rest[:tk]
        w_hbm = rest[tk:2 * tk]
        out_hbm = rest[2 * tk]
        sc = rest[2 * tk + 1:]
        idx_v = sc[0:5 * tk:5]
        w_v = sc[1:5 * tk:5]
        buf_v = sc[2:5 * tk:5]
        gsem = [sc[5 * k + 3:5 * k + 5] for k in range(tk)]
        wsem = sc[5 * tk:]
        wid = lax.axis_index("s") * _NC + lax.axis_index("c")
        base = wid * per_w
        for k in range(tk):
            pltpu.sync_copy(pos_hbm[k].at[pl.ds(base, per_w)], idx_v[k])
            pltpu.sync_copy(w_hbm[k].at[pl.ds(base, per_w)], w_v[k])

        def start_wave(c):
            par = c % 2
            return [pltpu.async_copy(
                y_hbm.at[idx_v[k].at[pl.ds(c * chunk, chunk)]],
                buf_v[k].at[par], gsem[k][par]) for k in range(tk)]

        def add_wave(c):
            par = c % 2

            def row_body(r, _):
                wk = [w_v[k][c * chunk + r, :] for k in range(tk)]

                def col_body(ci, _):
                    off = ci * 64
                    for s in range(4):
                        sl = pl.ds(off + s * 16, 16)
                        acc = buf_v[0][par, r, sl] * wk[0]
                        for k in range(1, tk):
                            acc = acc + buf_v[k][par, r, sl] * wk[k]
                        buf_v[0][par, r, sl] = acc
                    return 0

                return lax.fori_loop(0, d // 64, col_body, 0)

            lax.fori_loop(0, chunk, row_body, 0)
            return pltpu.async_copy(
                buf_v[0].at[par], out_hbm.at[pl.ds(base + c * chunk, chunk)],
                wsem[par])

        gq = [None] * n_chunks
        wq = [None] * n_chunks
        gq[0] = start_wave(0)
        for c in range(n_chunks):
            if c + 1 < n_chunks:
                if c >= 1:
                    wq[c - 1].wait()
                gq[c + 1] = start_wave(c + 1)
            for cp in gq[c]:
                cp.wait()
            wq[c] = add_wave(c)
        for c in range(max(0, n_chunks - 2), n_chunks):
            wq[c].wait()

    return combine_kernel(y, *pos_cols, *w_cols)


def kernel(x, expert_ids, expert_weights, gate_weights, up_weights,
           down_weights):
    nt, d = x.shape
    tk = expert_ids.shape[1]
    num_experts = gate_weights.shape[0]
    p = nt * tk
    cap = p + num_experts * _BM

    slot, block_expert = _dispatch_meta(expert_ids, num_experts, cap, _BM)

    per_w = nt // _NW
    slot_3d = slot.reshape(_NW, per_w, tk).transpose(0, 2, 1).copy()
    pos2d = slot.reshape(nt, tk)
    pos_cols = [pos2d[:, k].copy() for k in range(tk)]
    w_cols = [jnp.broadcast_to(expert_weights[:, k][:, None], (nt, 16)).copy()
              for k in range(tk)]

    xs = _sc_dispatch(x, slot_3d, cap, tk)
    y = _tc_moe_mlp(xs, gate_weights, up_weights, down_weights, block_expert,
                    cap, _BM)
    return _sc_combine(y, pos_cols, w_cols, nt, d, tk)
